# baseline (speedup 1.0000x reference)
import functools

import jax
import jax.numpy as jnp
from jax import lax
from jax.experimental import pallas as pl
from jax.experimental.pallas import tpu as pltpu
from jax.experimental.pallas import tpu_sc as plsc

_NC = 2
_NS = 16
_CH = 32


def kernel(x, pos_table):
    batch, seq_len = x.shape[0], x.shape[1]
    d_model = pos_table.shape[1]
    nw = _NC * _NS
    rows_per_w = seq_len // nw
    n_chunks = rows_per_w // _CH

    mesh = plsc.VectorSubcoreMesh(
        core_axis_name="c",
        subcore_axis_name="s",
        num_cores=_NC,
        num_subcores=_NS,
    )

    @functools.partial(
        pl.kernel,
        out_type=jax.ShapeDtypeStruct((batch, seq_len, d_model), jnp.float32),
        mesh=mesh,
        scratch_types=[
            pltpu.VMEM((_CH, d_model), jnp.float32),
            pltpu.VMEM((_CH, d_model), jnp.float32),
            pltpu.SemaphoreType.DMA,
            pltpu.SemaphoreType.DMA,
            pltpu.SemaphoreType.DMA,
            pltpu.SemaphoreType.DMA,
        ],
    )
    def broadcast_rows(table_hbm, out_hbm, buf0, buf1, rs0, rs1, ws0, ws1):
        bufs, rsems, wsems = [buf0, buf1], [rs0, rs1], [ws0, ws1]
        wid = lax.axis_index("s") * _NC + lax.axis_index("c")
        base = wid * rows_per_w

        rd = [None, None]
        wr = [[], []]
        rd[0] = pltpu.async_copy(table_hbm.at[pl.ds(base, _CH)], buf0, rs0)
        for g in range(n_chunks):
            cur = g & 1
            r0 = base + g * _CH
            rd[cur].wait()
            wr[cur] = [
                pltpu.async_copy(bufs[cur], out_hbm.at[b, pl.ds(r0, _CH)], wsems[cur])
                for b in range(batch)
            ]
            nxt = 1 - cur
            if g + 1 < n_chunks:
                for w in wr[nxt]:
                    w.wait()
                wr[nxt] = []
                r1 = base + (g + 1) * _CH
                rd[nxt] = pltpu.async_copy(
                    table_hbm.at[pl.ds(r1, _CH)], bufs[nxt], rsems[nxt])
        for lst in wr:
            for w in lst:
                w.wait()

    return broadcast_rows(pos_table)


# speedup vs baseline: 1.0485x; 1.0485x over previous
import functools

import jax
import jax.numpy as jnp
from jax import lax
from jax.experimental import pallas as pl
from jax.experimental.pallas import tpu as pltpu
from jax.experimental.pallas import tpu_sc as plsc

_NC = 2
_NS = 16
_CH = 64


def kernel(x, pos_table):
    batch, seq_len = x.shape[0], x.shape[1]
    d_model = pos_table.shape[1]
    nw = _NC * _NS
    rows_per_w = seq_len // nw
    n_chunks = rows_per_w // _CH

    mesh = plsc.VectorSubcoreMesh(
        core_axis_name="c",
        subcore_axis_name="s",
        num_cores=_NC,
        num_subcores=_NS,
    )

    @functools.partial(
        pl.kernel,
        out_type=jax.ShapeDtypeStruct((batch, seq_len, d_model), jnp.float32),
        mesh=mesh,
        scratch_types=[
            pltpu.VMEM((_CH, d_model), jnp.float32),
            pltpu.SemaphoreType.DMA,
            pltpu.SemaphoreType.DMA,
        ],
    )
    def broadcast_rows(table_hbm, out_hbm, buf, rsem, wsem):
        wid = lax.axis_index("s") * _NC + lax.axis_index("c")
        base = wid * rows_per_w

        for g in range(n_chunks):
            r0 = base + g * _CH
            pltpu.async_copy(table_hbm.at[pl.ds(r0, _CH)], buf, rsem).wait()
            writes = [
                pltpu.async_copy(buf, out_hbm.at[b, pl.ds(r0, _CH)], wsem)
                for b in range(batch)
            ]
            for w in writes:
                w.wait()

    return broadcast_rows(pos_table)


# speedup vs baseline: 1.0575x; 1.0086x over previous
import functools

import jax
import jax.numpy as jnp
from jax import lax
from jax.experimental import pallas as pl
from jax.experimental.pallas import tpu as pltpu
from jax.experimental.pallas import tpu_sc as plsc

_NC = 2
_NS = 16
_CH = 64


def kernel(x, pos_table):
    batch, seq_len = x.shape[0], x.shape[1]
    d_model = pos_table.shape[1]
    nw = _NC * _NS
    rows_per_w = seq_len // nw
    n_chunks = rows_per_w // _CH

    mesh = plsc.VectorSubcoreMesh(
        core_axis_name="c",
        subcore_axis_name="s",
        num_cores=_NC,
        num_subcores=_NS,
    )

    @functools.partial(
        pl.kernel,
        out_type=jax.ShapeDtypeStruct((batch, seq_len, d_model), jnp.float32),
        mesh=mesh,
        scratch_types=[
            pltpu.VMEM((_CH, d_model), jnp.float32),
            pltpu.SemaphoreType.DMA,
            pltpu.SemaphoreType.DMA,
        ],
    )
    def broadcast_rows(table_hbm, out_hbm, buf, rsem, wsem):
        wid = lax.axis_index("s") * _NC + lax.axis_index("c")
        base = wid * rows_per_w

        for g in range(n_chunks):
            r0 = base + g * _CH
            pltpu.async_copy(table_hbm.at[pl.ds(r0, _CH)], buf, rsem).wait()
            for b in range(batch):
                pltpu.sync_copy(buf, out_hbm.at[b, pl.ds(r0, _CH)])

    return broadcast_rows(pos_table)
